# baseline (speedup 1.0000x reference)
import functools

import jax
import jax.numpy as jnp
from jax import lax
from jax.experimental import pallas as pl
from jax.experimental.pallas import tpu as pltpu
from jax.experimental.pallas import tpu_sc as plsc


def _make_gather(n_total, vocab, dim, n_workers, num_cores, chunk):
    n_per_w = n_total // n_workers
    n_chunks = n_per_w // chunk
    mesh = plsc.VectorSubcoreMesh(core_axis_name="c", subcore_axis_name="s")

    @functools.partial(
        pl.kernel,
        mesh=mesh,
        out_type=jax.ShapeDtypeStruct((n_total, dim), jnp.float32),
        scratch_types=[
            pltpu.VMEM((n_per_w,), jnp.int32),
            pltpu.VMEM((2, chunk, dim), jnp.float32),
            pltpu.SemaphoreType.DMA,
            pltpu.SemaphoreType.DMA,
        ],
    )
    def emb(table_hbm, idx_hbm, out_hbm, idx_v, rows_v, gsem, wsem):
        wid = lax.axis_index("s") * num_cores + lax.axis_index("c")
        base = wid * n_per_w
        pltpu.sync_copy(idx_hbm.at[pl.ds(base, n_per_w)], idx_v)

        def start_gather(i):
            return pltpu.async_copy(
                table_hbm.at[idx_v.at[pl.ds(i * chunk, chunk)]],
                rows_v.at[i % 2],
                gsem,
            )

        gathers = [None] * n_chunks
        writes = [None] * n_chunks
        gathers[0] = start_gather(0)
        for i in range(n_chunks):
            if i + 1 < n_chunks:
                if i >= 1:
                    writes[i - 1].wait()
                gathers[i + 1] = start_gather(i + 1)
            gathers[i].wait()
            writes[i] = pltpu.async_copy(
                rows_v.at[i % 2],
                out_hbm.at[pl.ds(base + i * chunk, chunk)],
                wsem,
            )
        if n_chunks >= 2:
            writes[n_chunks - 2].wait()
        writes[n_chunks - 1].wait()

    return emb


def kernel(x, table):
    batch, hist = x.shape
    vocab, dim = table.shape
    hist_pad = 56
    n_total = batch * hist_pad

    xi = x.astype(jnp.int32)
    idx = jnp.concatenate(
        [xi, jnp.zeros((batch, hist_pad - hist), jnp.int32)], axis=1
    ).reshape(n_total)

    info = plsc.get_sparse_core_info()
    n_workers = info.num_cores * info.num_subcores
    chunk = 448

    emb = _make_gather(n_total, vocab, dim, n_workers, info.num_cores, chunk)
    rows = emb(table, idx)
    out_pad = rows.reshape(batch, hist_pad, dim)
    return lax.slice(out_pad, (0, 0, 0), (batch, hist, dim))


# speedup vs baseline: 7.7663x; 7.7663x over previous
import functools

import jax
import jax.numpy as jnp
from jax import lax
from jax.experimental import pallas as pl
from jax.experimental.pallas import tpu as pltpu
from jax.experimental.pallas import tpu_sc as plsc


def _make_gather(batch, hist, vocab, dim, n_workers, num_cores, rows_chunk):
    n_total = batch * hist
    n_per_w = n_total // n_workers
    b_per_w = batch // n_workers
    chunk = rows_chunk * hist
    n_chunks = b_per_w // rows_chunk
    mesh = plsc.VectorSubcoreMesh(core_axis_name="c", subcore_axis_name="s")

    @functools.partial(
        pl.kernel,
        mesh=mesh,
        out_type=jax.ShapeDtypeStruct((batch, hist, dim), jnp.float32),
        scratch_types=[
            pltpu.VMEM((n_per_w,), jnp.int32),
            pltpu.VMEM((2, chunk, dim), jnp.float32),
            pltpu.SemaphoreType.DMA,
            pltpu.SemaphoreType.DMA,
        ],
    )
    def emb(table_hbm, idx_hbm, out_hbm, idx_v, rows_v, gsem, wsem):
        wid = lax.axis_index("s") * num_cores + lax.axis_index("c")
        base = wid * n_per_w
        brow = wid * b_per_w
        pltpu.sync_copy(idx_hbm.at[pl.ds(base, n_per_w)], idx_v)

        def start_gather(i):
            return pltpu.async_copy(
                table_hbm.at[idx_v.at[pl.ds(i * chunk, chunk)]],
                rows_v.at[i % 2],
                gsem,
            )

        gathers = [None] * n_chunks
        writes = [None] * n_chunks
        gathers[0] = start_gather(0)
        for i in range(n_chunks):
            if i + 1 < n_chunks:
                if i >= 1:
                    writes[i - 1].wait()
                gathers[i + 1] = start_gather(i + 1)
            gathers[i].wait()
            writes[i] = pltpu.async_copy(
                rows_v.at[i % 2].reshape(rows_chunk, hist, dim),
                out_hbm.at[pl.ds(brow + i * rows_chunk, rows_chunk)],
                wsem,
            )
        if n_chunks >= 2:
            writes[n_chunks - 2].wait()
        writes[n_chunks - 1].wait()

    return emb


def kernel(x, table):
    batch, hist = x.shape
    vocab, dim = table.shape
    idx = x.reshape(batch * hist).astype(jnp.int32)

    info = plsc.get_sparse_core_info()
    n_workers = info.num_cores * info.num_subcores
    rows_chunk = 8

    emb = _make_gather(
        batch, hist, vocab, dim, n_workers, info.num_cores, rows_chunk
    )
    return emb(table, idx)
